# baseline (speedup 1.0000x reference)
import jax
import jax.numpy as jnp
from jax import lax
from jax.experimental import pallas as pl
from jax.experimental.pallas import tpu as pltpu
from jax.experimental.pallas import tpu_sc as plsc

PIECE_HEX_DIM = 120
P1_FEATURE_CUTOFF = 60
DENSE_DIM = 66
HEX_COUNT = 60
FT_DIM = 256
HIDDEN_DIM = 512
B = 16384

LANE = 128
DPAD = 72
R = 1024
OUT_ROWS = 64

NC = 2
NS = 16
NW = NC * NS
RPW = B // NW
GRP = 16
NGRP = RPW // GRP


def _sc_select(sp_hbm, out_hbm, sp_v, idx_v):
    wid = lax.axis_index("s") * NC + lax.axis_index("c")
    base = wid * RPW
    pltpu.sync_copy(sp_hbm.at[:, pl.ds(base, RPW)], sp_v)

    def group(g, _):
        cs = pl.ds(g * GRP, GRP)
        zeros = jnp.zeros((GRP,), jnp.int32)
        stm0 = jnp.where(sp_v[6, cs] == 0, 1, 0)
        cum = [zeros, zeros]
        slots = [[zeros, zeros, zeros], [zeros, zeros, zeros]]
        for j in range(6):
            spj = sp_v[j, cs]
            p1 = jnp.where(spj < P1_FEATURE_CUTOFF, 1, 0)
            same = stm0 * p1 + (1 - stm0) * (1 - p1)
            m = [same, 1 - same]
            for t in range(2):
                for s in range(3):
                    pick = m[t] * jnp.where(cum[t] == s, 1, 0)
                    slots[t][s] = slots[t][s] + pick * spj
                cum[t] = cum[t] + m[t]
        for t in range(2):
            for s in range(3):
                idx_v[3 * t + s, cs] = slots[t][s].astype(jnp.float32)
        return _

    jax.lax.fori_loop(0, NGRP, group, None)
    pltpu.sync_copy(idx_v, out_hbm.at[:, pl.ds(base, RPW)])


@jax.jit
def _run_sc(sp8):
    mesh = plsc.VectorSubcoreMesh(core_axis_name="c", subcore_axis_name="s")
    f = pl.kernel(
        _sc_select,
        mesh=mesh,
        out_type=jax.ShapeDtypeStruct((8, B), jnp.float32),
        scratch_types=[
            pltpu.VMEM((8, RPW), jnp.int32),
            pltpu.VMEM((8, RPW), jnp.float32),
        ],
    )
    return f(sp8)


def _fwd(idx_ref, dense_ref, ftw_ref, ftb_ref,
         w1a_ref, w1b_ref, w1c_ref, b1_ref, w2_ref, b2_ref,
         pol_ref, val_ref):
    cols = idx_ref.shape[1]
    iota_bf = lax.broadcasted_iota(
        jnp.int32, (LANE, cols), 0).astype(jnp.bfloat16)
    one = jnp.ones((LANE, cols), jnp.bfloat16)
    zero = jnp.zeros((LANE, cols), jnp.bfloat16)

    counts = [None, None]
    for t in range(2):
        acc0 = zero
        acc1 = zero
        for s in range(3):
            spx = idx_ref[3 * t + s:3 * t + s + 1, :].astype(jnp.bfloat16)
            hit = jnp.where(spx == iota_bf, one, zero)
            if s % 2 == 0:
                acc0 = acc0 + hit
            else:
                acc1 = acc1 + hit
        counts[t] = acc0 + acc1

    ftw = ftw_ref[...]
    ftb = ftb_ref[...]
    acc_s = jnp.maximum(
        jnp.dot(ftw, counts[0], preferred_element_type=jnp.float32) + ftb, 0.0)
    acc_n = jnp.maximum(
        jnp.dot(ftw, counts[1], preferred_element_type=jnp.float32) + ftb, 0.0)

    h = jnp.dot(w1a_ref[...], acc_s.astype(jnp.bfloat16),
                preferred_element_type=jnp.float32)
    h = h + jnp.dot(w1b_ref[...], acc_n.astype(jnp.bfloat16),
                    preferred_element_type=jnp.float32)
    h = h + lax.dot_general(w1c_ref[...], dense_ref[...],
                            (((1,), (1,)), ((), ())),
                            preferred_element_type=jnp.float32)
    h = jnp.maximum(h + b1_ref[...], 0.0)

    out = jnp.dot(w2_ref[...], h, preferred_element_type=jnp.float32)
    out = out + b2_ref[...]
    out_t = out.T
    pol_ref[...] = out_t[:, :HEX_COUNT]
    val_ref[...] = jnp.tanh(out_t[:, HEX_COUNT:HEX_COUNT + 1])


@jax.jit
def _run_tc(idx_t, dense_r, ftw, ftb, w1a, w1b, w1c, b1, w2, b2):
    grid = (B // R,)
    col = lambda i: (0, i)
    row = lambda i: (i, 0)
    rep = lambda i: (0, 0)
    pol, val = pl.pallas_call(
        _fwd,
        grid=grid,
        in_specs=[
            pl.BlockSpec((8, R), col),
            pl.BlockSpec((R, DENSE_DIM), row),
            pl.BlockSpec((FT_DIM, LANE), rep),
            pl.BlockSpec((FT_DIM, 1), rep),
            pl.BlockSpec((HIDDEN_DIM, FT_DIM), rep),
            pl.BlockSpec((HIDDEN_DIM, FT_DIM), rep),
            pl.BlockSpec((HIDDEN_DIM, DENSE_DIM), rep),
            pl.BlockSpec((HIDDEN_DIM, 1), rep),
            pl.BlockSpec((OUT_ROWS, HIDDEN_DIM), rep),
            pl.BlockSpec((OUT_ROWS, 1), rep),
        ],
        out_specs=[
            pl.BlockSpec((R, HEX_COUNT), row),
            pl.BlockSpec((R, 1), row),
        ],
        out_shape=[
            jax.ShapeDtypeStruct((B, HEX_COUNT), jnp.float32),
            jax.ShapeDtypeStruct((B, 1), jnp.float32),
        ],
    )(idx_t, dense_r, ftw, ftb, w1a, w1b, w1c, b1, w2, b2)
    return pol, val


def kernel(sparse_batch, dense_batch, stm_players, ft_W, ft_b,
           fc1_W, fc1_b, fc2v_W, fc2v_b, fc2p_W, fc2p_b):
    sp8 = jnp.concatenate(
        [sparse_batch.astype(jnp.int32).T,
         stm_players.astype(jnp.int32).reshape(1, B),
         jnp.zeros((1, B), jnp.int32)], axis=0)

    idx_t = _run_sc(sp8)

    dense_r = dense_batch.astype(jnp.bfloat16)

    ftw = jnp.pad(ft_W, ((0, 0), (0, LANE - PIECE_HEX_DIM))
                  ).astype(jnp.bfloat16)
    ftb = ft_b.reshape(FT_DIM, 1)

    w1a = fc1_W[:, :FT_DIM].astype(jnp.bfloat16)
    w1b = fc1_W[:, FT_DIM:2 * FT_DIM].astype(jnp.bfloat16)
    w1c = fc1_W[:, 2 * FT_DIM:].astype(jnp.bfloat16)
    b1 = fc1_b.reshape(HIDDEN_DIM, 1)

    w2 = jnp.pad(jnp.concatenate([fc2p_W, fc2v_W], axis=0),
                 ((0, OUT_ROWS - HEX_COUNT - 1), (0, 0)))
    b2 = jnp.pad(jnp.concatenate([fc2p_b, fc2v_b], axis=0),
                 (0, OUT_ROWS - HEX_COUNT - 1)).reshape(OUT_ROWS, 1)

    pol, val = _run_tc(idx_t, dense_r, ftw, ftb, w1a, w1b, w1c, b1, w2, b2)
    return (pol, val[:, 0])


# speedup vs baseline: 1.2596x; 1.2596x over previous
import jax
import jax.numpy as jnp
from jax import lax
from jax.experimental import pallas as pl
from jax.experimental.pallas import tpu as pltpu
from jax.experimental.pallas import tpu_sc as plsc

PIECE_HEX_DIM = 120
P1_FEATURE_CUTOFF = 60
DENSE_DIM = 66
HEX_COUNT = 60
FT_DIM = 256
HIDDEN_DIM = 512
B = 16384

LANE = 128
DPAD = 72
R = 2048
OUT_ROWS = 64

NC = 2
NS = 16
NW = NC * NS
RPW = B // NW
GRP = 16
NGRP = RPW // GRP


def _sc_select(sp_hbm, out_hbm, sp_v, idx_v):
    wid = lax.axis_index("s") * NC + lax.axis_index("c")
    base = wid * RPW
    pltpu.sync_copy(sp_hbm.at[:, pl.ds(base, RPW)], sp_v)

    def group(g, _):
        cs = pl.ds(g * GRP, GRP)
        zeros = jnp.zeros((GRP,), jnp.int32)
        stm0 = jnp.where(sp_v[6, cs] == 0, 1, 0)
        cum = [zeros, zeros]
        slots = [[zeros, zeros, zeros], [zeros, zeros, zeros]]
        for j in range(6):
            spj = sp_v[j, cs]
            p1 = jnp.where(spj < P1_FEATURE_CUTOFF, 1, 0)
            same = stm0 * p1 + (1 - stm0) * (1 - p1)
            m = [same, 1 - same]
            for t in range(2):
                for s in range(3):
                    pick = m[t] * jnp.where(cum[t] == s, 1, 0)
                    slots[t][s] = slots[t][s] + pick * spj
                cum[t] = cum[t] + m[t]
        for t in range(2):
            for s in range(3):
                idx_v[3 * t + s, cs] = slots[t][s].astype(jnp.float32)
        return _

    jax.lax.fori_loop(0, NGRP, group, None)
    pltpu.sync_copy(idx_v, out_hbm.at[:, pl.ds(base, RPW)])


@jax.jit
def _run_sc(sp8):
    mesh = plsc.VectorSubcoreMesh(core_axis_name="c", subcore_axis_name="s")
    f = pl.kernel(
        _sc_select,
        mesh=mesh,
        out_type=jax.ShapeDtypeStruct((8, B), jnp.float32),
        scratch_types=[
            pltpu.VMEM((8, RPW), jnp.int32),
            pltpu.VMEM((8, RPW), jnp.float32),
        ],
    )
    return f(sp8)


def _fwd(idx_ref, dense_ref, ftw_ref, ftb_ref,
         w1a_ref, w1b_ref, w1c_ref, b1_ref, w2_ref, b2_ref,
         out_ref):
    cols = idx_ref.shape[1]
    iota_bf = lax.broadcasted_iota(
        jnp.int32, (LANE, cols), 0).astype(jnp.bfloat16)
    one = jnp.ones((LANE, cols), jnp.bfloat16)
    zero = jnp.zeros((LANE, cols), jnp.bfloat16)

    counts = [None, None]
    for t in range(2):
        acc0 = zero
        acc1 = zero
        for s in range(3):
            spx = idx_ref[3 * t + s:3 * t + s + 1, :].astype(jnp.bfloat16)
            hit = jnp.where(spx == iota_bf, one, zero)
            if s % 2 == 0:
                acc0 = acc0 + hit
            else:
                acc1 = acc1 + hit
        counts[t] = acc0 + acc1

    ftw = ftw_ref[...]
    ftb = ftb_ref[...]
    acc_s = jnp.maximum(
        jnp.dot(ftw, counts[0], preferred_element_type=jnp.float32) + ftb, 0.0)
    acc_n = jnp.maximum(
        jnp.dot(ftw, counts[1], preferred_element_type=jnp.float32) + ftb, 0.0)

    h = jnp.dot(w1a_ref[...], acc_s.astype(jnp.bfloat16),
                preferred_element_type=jnp.float32)
    h = h + jnp.dot(w1b_ref[...], acc_n.astype(jnp.bfloat16),
                    preferred_element_type=jnp.float32)
    h = h + jnp.dot(w1c_ref[...], dense_ref[...],
                    preferred_element_type=jnp.float32)
    h = jnp.maximum(h + b1_ref[...], 0.0)

    out = jnp.dot(w2_ref[...], h, preferred_element_type=jnp.float32)
    out = out + b2_ref[...]
    vrow = lax.broadcasted_iota(jnp.int32, (OUT_ROWS, cols), 0) == HEX_COUNT
    out_ref[...] = jnp.where(vrow, jnp.tanh(out), out)


@jax.jit
def _run_tc(idx_t, dense_t, ftw, ftb, w1a, w1b, w1c, b1, w2, b2):
    grid = (B // R,)
    col = lambda i: (0, i)
    rep = lambda i: (0, 0)
    out = pl.pallas_call(
        _fwd,
        grid=grid,
        in_specs=[
            pl.BlockSpec((8, R), col),
            pl.BlockSpec((DPAD, R), col),
            pl.BlockSpec((FT_DIM, LANE), rep),
            pl.BlockSpec((FT_DIM, 1), rep),
            pl.BlockSpec((HIDDEN_DIM, FT_DIM), rep),
            pl.BlockSpec((HIDDEN_DIM, FT_DIM), rep),
            pl.BlockSpec((HIDDEN_DIM, DPAD), rep),
            pl.BlockSpec((HIDDEN_DIM, 1), rep),
            pl.BlockSpec((OUT_ROWS, HIDDEN_DIM), rep),
            pl.BlockSpec((OUT_ROWS, 1), rep),
        ],
        out_specs=pl.BlockSpec((OUT_ROWS, R), col),
        out_shape=jax.ShapeDtypeStruct((OUT_ROWS, B), jnp.float32),
    )(idx_t, dense_t, ftw, ftb, w1a, w1b, w1c, b1, w2, b2)
    return out


def kernel(sparse_batch, dense_batch, stm_players, ft_W, ft_b,
           fc1_W, fc1_b, fc2v_W, fc2v_b, fc2p_W, fc2p_b):
    sp8 = jnp.concatenate(
        [sparse_batch.astype(jnp.int32).T,
         stm_players.astype(jnp.int32).reshape(1, B),
         jnp.zeros((1, B), jnp.int32)], axis=0)

    idx_t = _run_sc(sp8)

    dense_t = jnp.pad(dense_batch.T.astype(jnp.bfloat16),
                      ((0, DPAD - DENSE_DIM), (0, 0)))

    ftw = jnp.pad(ft_W, ((0, 0), (0, LANE - PIECE_HEX_DIM))
                  ).astype(jnp.bfloat16)
    ftb = ft_b.reshape(FT_DIM, 1)

    w1a = fc1_W[:, :FT_DIM].astype(jnp.bfloat16)
    w1b = fc1_W[:, FT_DIM:2 * FT_DIM].astype(jnp.bfloat16)
    w1c = jnp.pad(fc1_W[:, 2 * FT_DIM:],
                  ((0, 0), (0, DPAD - DENSE_DIM))).astype(jnp.bfloat16)
    b1 = fc1_b.reshape(HIDDEN_DIM, 1)

    w2 = jnp.pad(jnp.concatenate([fc2p_W, fc2v_W], axis=0),
                 ((0, OUT_ROWS - HEX_COUNT - 1), (0, 0)))
    b2 = jnp.pad(jnp.concatenate([fc2p_b, fc2v_b], axis=0),
                 (0, OUT_ROWS - HEX_COUNT - 1)).reshape(OUT_ROWS, 1)

    out = _run_tc(idx_t, dense_t, ftw, ftb, w1a, w1b, w1c, b1, w2, b2)
    return (out[:HEX_COUNT].T, out[HEX_COUNT])
